# baseline (speedup 1.0000x reference)
import functools

import jax
import jax.numpy as jnp
from jax import lax
from jax.experimental import pallas as pl
from jax.experimental.pallas import tpu as pltpu
from jax.experimental.pallas import tpu_sc as plsc

D = 1024
ROWS = 16384
NW = 32
ROWS_PER_W = ROWS // NW
C = 32
NCHUNK = ROWS_PER_W // C
LANES = 16

_mesh = plsc.VectorSubcoreMesh(core_axis_name="c", subcore_axis_name="s")


@functools.partial(
    pl.kernel,
    mesh=_mesh,
    out_type=jax.ShapeDtypeStruct((ROWS, D), jnp.float32),
    scratch_types=[
        pltpu.VMEM((NCHUNK, C), jnp.int32),
        pltpu.VMEM((C, D), jnp.float32),
        pltpu.VMEM((C, D), jnp.float32),
        pltpu.SemaphoreType.DMA,
    ],
)
def _sc_kernel(x_hbm, idx_hbm, pe_hbm, out_hbm, idx_v, xbuf, pebuf, sem):
    wid = lax.axis_index("s") * 2 + lax.axis_index("c")
    base = wid * ROWS_PER_W
    pltpu.sync_copy(idx_hbm.at[wid], idx_v)

    def chunk(c, carry):
        off = base + c * C
        pltpu.sync_copy(x_hbm.at[pl.ds(off, C)], xbuf)
        pltpu.async_copy(pe_hbm.at[idx_v.at[c]], pebuf, sem).wait()

        def row(j, carry2):
            def lane(i, carry3):
                sl = pl.ds(i * LANES, LANES)
                xbuf[j, sl] = xbuf[j, sl] + pebuf[j, sl]
                return carry3
            return lax.fori_loop(0, D // LANES, lane, carry2)

        lax.fori_loop(0, C, row, carry)
        pltpu.sync_copy(xbuf, out_hbm.at[pl.ds(off, C)])
        return carry

    lax.fori_loop(0, NCHUNK, chunk, 0)


def kernel(x, pos_id_torch_pad, pe_table):
    xf = x.reshape(ROWS, D)
    idx = pos_id_torch_pad.astype(jnp.int32).reshape(NW, NCHUNK, C)
    out = _sc_kernel(xf, idx, pe_table)
    return out.reshape(x.shape)


# speedup vs baseline: 2.1842x; 2.1842x over previous
import functools

import jax
import jax.numpy as jnp
from jax import lax
from jax.experimental import pallas as pl
from jax.experimental.pallas import tpu as pltpu
from jax.experimental.pallas import tpu_sc as plsc

D = 1024
ROWS = 16384
NW = 32
ROWS_PER_W = ROWS // NW
C = 16
NCHUNK = ROWS_PER_W // C
NBUF = 2
LANES = 16

_mesh = plsc.VectorSubcoreMesh(core_axis_name="c", subcore_axis_name="s")


@functools.partial(
    pl.kernel,
    mesh=_mesh,
    out_type=jax.ShapeDtypeStruct((ROWS, D), jnp.float32),
    scratch_types=[
        pltpu.VMEM((NCHUNK, C), jnp.int32),
        pltpu.VMEM((NBUF, C, D), jnp.float32),
        pltpu.VMEM((NBUF, C, D), jnp.float32),
        pltpu.VMEM((NBUF, C, D), jnp.float32),
        pltpu.SemaphoreType.DMA,
        pltpu.SemaphoreType.DMA,
        pltpu.SemaphoreType.DMA,
        pltpu.SemaphoreType.DMA,
    ],
)
def _sc_kernel(x_hbm, idx_hbm, pe_hbm, out_hbm,
               idx_v, xbuf, pebuf, obuf, si0, si1, so0, so1):
    sem_in = (si0, si1)
    sem_out = (so0, so1)
    wid = lax.axis_index("s") * 2 + lax.axis_index("c")
    base = wid * ROWS_PER_W
    pltpu.sync_copy(idx_hbm.at[wid], idx_v)

    def start_in(c, b):
        off = base + c * C
        pltpu.async_copy(x_hbm.at[pl.ds(off, C)], xbuf.at[b], sem_in[b])
        pltpu.async_copy(pe_hbm.at[idx_v.at[c]], pebuf.at[b], sem_in[b])

    def wait_in(b):
        pltpu.make_async_copy(x_hbm.at[pl.ds(0, C)], xbuf.at[b], sem_in[b]).wait()
        pltpu.make_async_copy(x_hbm.at[pl.ds(0, C)], pebuf.at[b], sem_in[b]).wait()

    def wait_out(b):
        pltpu.make_async_copy(x_hbm.at[pl.ds(0, C)], obuf.at[b], sem_out[b]).wait()

    for b in range(NBUF):
        start_in(b, b)

    @pl.loop(0, NCHUNK, step=NBUF)
    def _outer(o):
        for b in range(NBUF):
            c = o + b
            @pl.when(o >= NBUF)
            def _():
                wait_out(b)
            wait_in(b)

            @pl.loop(0, C)
            def _row(j):
                for g in range(D // LANES):
                    sl = pl.ds(g * LANES, LANES)
                    obuf[b, j, sl] = xbuf[b, j, sl] + pebuf[b, j, sl]

            pltpu.async_copy(obuf.at[b], out_hbm.at[pl.ds(base + c * C, C)],
                             sem_out[b])

            @pl.when(o < NCHUNK - NBUF)
            def _():
                start_in(c + NBUF, b)

    for b in range(NBUF):
        wait_out(b)


def kernel(x, pos_id_torch_pad, pe_table):
    xf = x.reshape(ROWS, D)
    idx = pos_id_torch_pad.astype(jnp.int32).reshape(NW, NCHUNK, C)
    out = _sc_kernel(xf, idx, pe_table)
    return out.reshape(x.shape)


# speedup vs baseline: 2.7174x; 1.2441x over previous
import functools

import jax
import jax.numpy as jnp
from jax import lax
from jax.experimental import pallas as pl
from jax.experimental.pallas import tpu as pltpu
from jax.experimental.pallas import tpu_sc as plsc

D = 1024
ROWS = 16384
NW = 32
ROWS_PER_W = ROWS // NW
C = 8
NCHUNK = ROWS_PER_W // C
NBUF = 4
LANES = 16

_mesh = plsc.VectorSubcoreMesh(core_axis_name="c", subcore_axis_name="s")


@functools.partial(
    pl.kernel,
    mesh=_mesh,
    out_type=jax.ShapeDtypeStruct((ROWS, D), jnp.float32),
    scratch_types=[
        pltpu.VMEM((NCHUNK, C), jnp.int32),
        pltpu.VMEM((NBUF, C, D), jnp.float32),
        pltpu.VMEM((NBUF, C, D), jnp.float32),
        pltpu.VMEM((NBUF, C, D), jnp.float32),
        pltpu.SemaphoreType.DMA,
        pltpu.SemaphoreType.DMA,
        pltpu.SemaphoreType.DMA,
        pltpu.SemaphoreType.DMA,
        pltpu.SemaphoreType.DMA,
        pltpu.SemaphoreType.DMA,
        pltpu.SemaphoreType.DMA,
        pltpu.SemaphoreType.DMA,
    ],
)
def _sc_kernel(x_hbm, idx_hbm, pe_hbm, out_hbm,
               idx_v, xbuf, pebuf, obuf, si0, si1, si2, si3, so0, so1, so2, so3):
    sem_in = (si0, si1, si2, si3)
    sem_out = (so0, so1, so2, so3)
    wid = lax.axis_index("s") * 2 + lax.axis_index("c")
    base = wid * ROWS_PER_W
    pltpu.sync_copy(idx_hbm.at[wid], idx_v)

    def start_in(c, b):
        off = base + c * C
        pltpu.async_copy(x_hbm.at[pl.ds(off, C)], xbuf.at[b], sem_in[b])
        pltpu.async_copy(pe_hbm.at[idx_v.at[c]], pebuf.at[b], sem_in[b])

    def wait_in(b):
        pltpu.make_async_copy(x_hbm.at[pl.ds(0, C)], xbuf.at[b], sem_in[b]).wait()
        pltpu.make_async_copy(x_hbm.at[pl.ds(0, C)], pebuf.at[b], sem_in[b]).wait()

    def wait_out(b):
        pltpu.make_async_copy(x_hbm.at[pl.ds(0, C)], obuf.at[b], sem_out[b]).wait()

    for b in range(NBUF):
        start_in(b, b)

    @pl.loop(0, NCHUNK, step=NBUF)
    def _outer(o):
        for b in range(NBUF):
            c = o + b
            @pl.when(o >= NBUF)
            def _():
                wait_out(b)
            wait_in(b)

            @pl.loop(0, C)
            def _row(j):
                for g in range(D // LANES):
                    sl = pl.ds(g * LANES, LANES)
                    obuf[b, j, sl] = xbuf[b, j, sl] + pebuf[b, j, sl]

            pltpu.async_copy(obuf.at[b], out_hbm.at[pl.ds(base + c * C, C)],
                             sem_out[b])

            @pl.when(o < NCHUNK - NBUF)
            def _():
                start_in(c + NBUF, b)

    for b in range(NBUF):
        wait_out(b)


def kernel(x, pos_id_torch_pad, pe_table):
    xf = x.reshape(ROWS, D)
    idx = pos_id_torch_pad.astype(jnp.int32).reshape(NW, NCHUNK, C)
    out = _sc_kernel(xf, idx, pe_table)
    return out.reshape(x.shape)
